# Initial kernel scaffold; baseline (speedup 1.0000x reference)
import jax
import jax.numpy as jnp
from jax.experimental import pallas as pl


def kernel(partial_cloud, kp, W1, b1, W2, b2):
    raise NotImplementedError("write your pallas kernel here")


# speedup vs baseline: 20.6200x; 20.6200x over previous
import functools

import jax
import jax.numpy as jnp
from jax import lax
from jax.experimental import pallas as pl
from jax.experimental.pallas import tpu as pltpu
from jax.experimental.pallas import tpu_sc as plsc

B = 8
N = 8320
S = 512
K = 16
QCHUNK = 128
NQC = S // QCHUNK
NW = 32
CHUNK = (B * S * K) // NW
GW = 128
NCH = CHUNK // GW


def _fps_body(x_ref, y_ref, z_ref, cx_ref, cy_ref, cz_ref, bn_ref, dist_ref):
    x = x_ref[...]
    y = y_ref[...]
    z = z_ref[...]
    bn_ref[...] = (x * x + y * y) + z * z

    lane_iota = lax.broadcasted_iota(jnp.int32, (B, N), 1)
    sel_iota = lax.broadcasted_iota(jnp.int32, (B, S), 1)
    cx_ref[...] = jnp.zeros((B, S), jnp.float32)
    cy_ref[...] = jnp.zeros((B, S), jnp.float32)
    cz_ref[...] = jnp.zeros((B, S), jnp.float32)
    dist_ref[...] = jnp.full((B, N), 1e10, jnp.float32)

    def body(i, far):
        m2 = lane_iota == far
        cx = jnp.sum(jnp.where(m2, x, 0.0), axis=1, keepdims=True)
        cy = jnp.sum(jnp.where(m2, y, 0.0), axis=1, keepdims=True)
        cz = jnp.sum(jnp.where(m2, z, 0.0), axis=1, keepdims=True)
        onehot = sel_iota == i
        cx_ref[...] += jnp.where(onehot, cx, 0.0)
        cy_ref[...] += jnp.where(onehot, cy, 0.0)
        cz_ref[...] += jnp.where(onehot, cz, 0.0)
        dx = x - cx
        dy = y - cy
        dz = z - cz
        dd = (dx * dx + dy * dy) + dz * dz
        dist = jnp.minimum(dist_ref[...], dd)
        dist_ref[...] = dist
        mx = jnp.max(dist, axis=1, keepdims=True)
        far_new = jnp.min(jnp.where(dist == mx, lane_iota, N),
                          axis=1, keepdims=True).astype(jnp.int32)
        return far_new

    lax.fori_loop(0, S, body, jnp.zeros((B, 1), jnp.int32))


def _run_fps(x, y, z):
    return pl.pallas_call(
        _fps_body,
        out_shape=[
            jax.ShapeDtypeStruct((B, S), jnp.float32),
            jax.ShapeDtypeStruct((B, S), jnp.float32),
            jax.ShapeDtypeStruct((B, S), jnp.float32),
            jax.ShapeDtypeStruct((B, N), jnp.float32),
        ],
        scratch_shapes=[pltpu.VMEM((B, N), jnp.float32)],
    )(x, y, z)


def _knn_body(x_ref, y_ref, z_ref, bn_ref, cx_ref, cy_ref, cz_ref,
              idx_ref, dist_ref):
    Xr = jnp.concatenate([x_ref[0], y_ref[0], z_ref[0]], axis=0)
    QT = jnp.concatenate([cx_ref[0], cy_ref[0], cz_ref[0]], axis=0)
    Q = jnp.transpose(QT, (1, 0))
    C = lax.dot_general(Q.astype(jnp.bfloat16), Xr.astype(jnp.bfloat16),
                        (((1,), (0,)), ((), ())),
                        preferred_element_type=jnp.float32)
    qx = Q[:, 0:1]
    qy = Q[:, 1:2]
    qz = Q[:, 2:3]
    A = (qx * qx + qy * qy) + qz * qz
    dist_ref[...] = (A + bn_ref[0]) - 2.0 * C

    iota = lax.broadcasted_iota(jnp.int32, (QCHUNK, N), 1)

    def pass_body(p, carry):
        prev, acc = carry
        d = jnp.where(iota == prev, jnp.inf, dist_ref[...])
        dist_ref[...] = d
        m = jnp.min(d, axis=1, keepdims=True)
        idxc = jnp.min(jnp.where(d == m, iota, N), axis=1, keepdims=True)
        kio = lax.broadcasted_iota(jnp.int32, (QCHUNK, K), 1)
        acc = jnp.where(kio == p, idxc, acc)
        return idxc, acc

    _, acc = lax.fori_loop(
        0, K, pass_body,
        (jnp.full((QCHUNK, 1), -1, jnp.int32),
         jnp.zeros((QCHUNK, K), jnp.int32)))
    idx_ref[0] = acc + pl.program_id(0) * N


def _run_knn(x3, y3, z3, bn3, cxr, cyr, czr):
    plane = pl.BlockSpec((1, 1, N), lambda b, q: (b, 0, 0))
    cspec = pl.BlockSpec((1, 1, QCHUNK), lambda b, q: (b * NQC + q, 0, 0))
    return pl.pallas_call(
        _knn_body,
        grid=(B, NQC),
        in_specs=[plane, plane, plane, plane, cspec, cspec, cspec],
        out_specs=pl.BlockSpec((1, QCHUNK, K), lambda b, q: (b * NQC + q, 0, 0)),
        out_shape=jax.ShapeDtypeStruct((B * NQC, QCHUNK, K), jnp.int32),
        scratch_shapes=[pltpu.VMEM((QCHUNK, N), jnp.float32)],
    )(x3, y3, z3, bn3, cxr, cyr, czr)


def _sc_gather_body(xs_hbm, ys_hbm, zs_hbm, idx_hbm,
                    ogx_hbm, ogy_hbm, ogz_hbm,
                    idxv, gxv, gyv, gzv, sem):
    wid = lax.axis_index("s") * 2 + lax.axis_index("c")
    pltpu.sync_copy(idx_hbm.at[wid], idxv)
    copies = []
    for j in range(NCH):
        iv = idxv.at[j]
        sl = pl.ds(j * GW, GW)
        copies.append(pltpu.async_copy(xs_hbm.at[iv], gxv.at[sl], sem))
        copies.append(pltpu.async_copy(ys_hbm.at[iv], gyv.at[sl], sem))
        copies.append(pltpu.async_copy(zs_hbm.at[iv], gzv.at[sl], sem))
    for c in copies:
        c.wait()
    pltpu.sync_copy(gxv, ogx_hbm.at[wid])
    pltpu.sync_copy(gyv, ogy_hbm.at[wid])
    pltpu.sync_copy(gzv, ogz_hbm.at[wid])


def _run_sc_gather(xs, ys, zs, idx):
    f32 = jnp.float32
    out = jax.ShapeDtypeStruct((NW, CHUNK), f32)
    fn = functools.partial(
        pl.kernel,
        mesh=plsc.VectorSubcoreMesh(core_axis_name="c", subcore_axis_name="s"),
        out_type=[out] * 3,
        scratch_types=[
            pltpu.VMEM((NCH, GW), jnp.int32),
            pltpu.VMEM((CHUNK,), f32),
            pltpu.VMEM((CHUNK,), f32),
            pltpu.VMEM((CHUNK,), f32),
            pltpu.SemaphoreType.DMA,
        ],
    )(_sc_gather_body)
    return fn(xs, ys, zs, idx)


def _mlp_body(gx_ref, gy_ref, gz_ref, cx_ref, cy_ref, cz_ref,
              w1g_ref, w1c_ref, b1_ref, w2_ref, b2_ref, out_ref):
    G = jnp.concatenate([gx_ref[0], gy_ref[0], gz_ref[0]], axis=0)
    Cm = jnp.concatenate([cx_ref[0], cy_ref[0], cz_ref[0]], axis=0)
    A = lax.dot_general(w1g_ref[...].astype(jnp.bfloat16),
                        G.astype(jnp.bfloat16),
                        (((1,), (0,)), ((), ())),
                        preferred_element_type=jnp.float32)
    T2 = lax.dot_general(w1c_ref[...].astype(jnp.bfloat16),
                         Cm.astype(jnp.bfloat16),
                         (((1,), (0,)), ((), ())),
                         preferred_element_type=jnp.float32)
    M = b1_ref[...] - T2
    w2 = w2_ref[...].astype(jnp.bfloat16)
    acc = None
    for k in range(K):
        h = jnp.maximum(A[:, k * S:(k + 1) * S] + M, 0.0)
        h2 = lax.dot_general(w2, h.astype(jnp.bfloat16),
                             (((1,), (0,)), ((), ())),
                             preferred_element_type=jnp.float32)
        acc = h2 if acc is None else jnp.maximum(acc, h2)
    out_ref[0] = acc + b2_ref[...]


def _run_mlp(gx3, gy3, gz3, cx3, cy3, cz3, W1g, W1c, b1r, W2, b2r):
    row = pl.BlockSpec((1, 1, S * K), lambda b: (b, 0, 0))
    crow = pl.BlockSpec((1, 1, S), lambda b: (b, 0, 0))
    return pl.pallas_call(
        _mlp_body,
        grid=(B,),
        in_specs=[row, row, row, crow, crow, crow,
                  pl.BlockSpec((64, 3), lambda b: (0, 0)),
                  pl.BlockSpec((64, 3), lambda b: (0, 0)),
                  pl.BlockSpec((64, 1), lambda b: (0, 0)),
                  pl.BlockSpec((128, 64), lambda b: (0, 0)),
                  pl.BlockSpec((128, 1), lambda b: (0, 0))],
        out_specs=pl.BlockSpec((1, 128, S), lambda b: (b, 0, 0)),
        out_shape=jax.ShapeDtypeStruct((B, 128, S), jnp.float32),
    )(gx3, gy3, gz3, cx3, cy3, cz3, W1g, W1c, b1r, W2, b2r)


def kernel(partial_cloud, kp, W1, b1, W2, b2):
    xyz = jnp.concatenate([partial_cloud, kp], axis=2)
    x = xyz[:, 0, :]
    y = xyz[:, 1, :]
    z = xyz[:, 2, :]

    cx, cy, cz, bn = _run_fps(x, y, z)

    idx = _run_knn(
        x.reshape(B, 1, N), y.reshape(B, 1, N), z.reshape(B, 1, N),
        bn.reshape(B, 1, N),
        cx.reshape(B * NQC, 1, QCHUNK), cy.reshape(B * NQC, 1, QCHUNK),
        cz.reshape(B * NQC, 1, QCHUNK))

    idx_sc = (idx.reshape(B, NQC, QCHUNK, K)
              .transpose(0, 3, 1, 2)
              .reshape(NW, NCH, GW))

    gx, gy, gz = _run_sc_gather(
        x.reshape(B * N), y.reshape(B * N), z.reshape(B * N), idx_sc)

    W1a = W1[:, :3]
    out = _run_mlp(
        gx.reshape(B, 1, S * K), gy.reshape(B, 1, S * K),
        gz.reshape(B, 1, S * K),
        cx.reshape(B, 1, S), cy.reshape(B, 1, S), cz.reshape(B, 1, S),
        W1a + W1[:, 3:], W1a, b1.reshape(64, 1), W2, b2.reshape(128, 1))
    return out


# speedup vs baseline: 21.6864x; 1.0517x over previous
import functools

import jax
import jax.numpy as jnp
from jax import lax
from jax.experimental import pallas as pl
from jax.experimental.pallas import tpu as pltpu
from jax.experimental.pallas import tpu_sc as plsc

B = 8
N = 8320
S = 512
K = 16
QCHUNK = 128
NQC = S // QCHUNK
NW = 32
CHUNK = (B * S * K) // NW
GW = 128
NCH = CHUNK // GW


def _fps_body(x_ref, y_ref, z_ref, cx_ref, cy_ref, cz_ref, bn_ref, dist_ref):
    x = x_ref[...]
    y = y_ref[...]
    z = z_ref[...]
    bn_ref[...] = (x * x + y * y) + z * z

    lane_iota = lax.broadcasted_iota(jnp.int32, (B, N), 1)
    sel_iota = lax.broadcasted_iota(jnp.int32, (B, S), 1)
    cx_ref[...] = jnp.zeros((B, S), jnp.float32)
    cy_ref[...] = jnp.zeros((B, S), jnp.float32)
    cz_ref[...] = jnp.zeros((B, S), jnp.float32)
    dist_ref[...] = jnp.full((B, N), 1e10, jnp.float32)

    def body(i, far):
        m2 = lane_iota == far
        cx = jnp.sum(jnp.where(m2, x, 0.0), axis=1, keepdims=True)
        cy = jnp.sum(jnp.where(m2, y, 0.0), axis=1, keepdims=True)
        cz = jnp.sum(jnp.where(m2, z, 0.0), axis=1, keepdims=True)
        onehot = sel_iota == i
        cx_ref[...] += jnp.where(onehot, cx, 0.0)
        cy_ref[...] += jnp.where(onehot, cy, 0.0)
        cz_ref[...] += jnp.where(onehot, cz, 0.0)
        dx = x - cx
        dy = y - cy
        dz = z - cz
        dd = (dx * dx + dy * dy) + dz * dz
        dist = jnp.minimum(dist_ref[...], dd)
        dist_ref[...] = dist
        far_new = jnp.argmax(dist, axis=1).astype(jnp.int32)[:, None]
        return far_new

    lax.fori_loop(0, S, body, jnp.zeros((B, 1), jnp.int32))


def _run_fps(x, y, z):
    return pl.pallas_call(
        _fps_body,
        out_shape=[
            jax.ShapeDtypeStruct((B, S), jnp.float32),
            jax.ShapeDtypeStruct((B, S), jnp.float32),
            jax.ShapeDtypeStruct((B, S), jnp.float32),
            jax.ShapeDtypeStruct((B, N), jnp.float32),
        ],
        scratch_shapes=[pltpu.VMEM((B, N), jnp.float32)],
    )(x, y, z)


def _knn_body(x_ref, y_ref, z_ref, bn_ref, cx_ref, cy_ref, cz_ref,
              idx_ref, dist_ref):
    Xr = jnp.concatenate([x_ref[0], y_ref[0], z_ref[0]], axis=0)
    QT = jnp.concatenate([cx_ref[0], cy_ref[0], cz_ref[0]], axis=0)
    Q = jnp.transpose(QT, (1, 0))
    C = lax.dot_general(Q.astype(jnp.bfloat16), Xr.astype(jnp.bfloat16),
                        (((1,), (0,)), ((), ())),
                        preferred_element_type=jnp.float32)
    qx = Q[:, 0:1]
    qy = Q[:, 1:2]
    qz = Q[:, 2:3]
    A = (qx * qx + qy * qy) + qz * qz
    dist_ref[...] = 2.0 * C - (A + bn_ref[0])

    iota = lax.broadcasted_iota(jnp.int32, (QCHUNK, N), 1)

    def pass_body(p, carry):
        prev, acc = carry
        s = jnp.where(iota == prev, -jnp.inf, dist_ref[...])
        dist_ref[...] = s
        idxc = jnp.argmax(s, axis=1).astype(jnp.int32)[:, None]
        kio = lax.broadcasted_iota(jnp.int32, (QCHUNK, K), 1)
        acc = jnp.where(kio == p, idxc, acc)
        return idxc, acc

    _, acc = lax.fori_loop(
        0, K, pass_body,
        (jnp.full((QCHUNK, 1), -1, jnp.int32),
         jnp.zeros((QCHUNK, K), jnp.int32)))
    idx_ref[0] = acc + pl.program_id(0) * N


def _run_knn(x3, y3, z3, bn3, cxr, cyr, czr):
    plane = pl.BlockSpec((1, 1, N), lambda b, q: (b, 0, 0))
    cspec = pl.BlockSpec((1, 1, QCHUNK), lambda b, q: (b * NQC + q, 0, 0))
    return pl.pallas_call(
        _knn_body,
        grid=(B, NQC),
        in_specs=[plane, plane, plane, plane, cspec, cspec, cspec],
        out_specs=pl.BlockSpec((1, QCHUNK, K), lambda b, q: (b * NQC + q, 0, 0)),
        out_shape=jax.ShapeDtypeStruct((B * NQC, QCHUNK, K), jnp.int32),
        scratch_shapes=[pltpu.VMEM((QCHUNK, N), jnp.float32)],
    )(x3, y3, z3, bn3, cxr, cyr, czr)


def _sc_gather_body(xs_hbm, ys_hbm, zs_hbm, idx_hbm,
                    ogx_hbm, ogy_hbm, ogz_hbm,
                    idxv, gxv, gyv, gzv, sem):
    wid = lax.axis_index("s") * 2 + lax.axis_index("c")
    pltpu.sync_copy(idx_hbm.at[wid], idxv)
    copies = []
    for j in range(NCH):
        iv = idxv.at[j]
        sl = pl.ds(j * GW, GW)
        copies.append(pltpu.async_copy(xs_hbm.at[iv], gxv.at[sl], sem))
        copies.append(pltpu.async_copy(ys_hbm.at[iv], gyv.at[sl], sem))
        copies.append(pltpu.async_copy(zs_hbm.at[iv], gzv.at[sl], sem))
    for c in copies:
        c.wait()
    pltpu.sync_copy(gxv, ogx_hbm.at[wid])
    pltpu.sync_copy(gyv, ogy_hbm.at[wid])
    pltpu.sync_copy(gzv, ogz_hbm.at[wid])


def _run_sc_gather(xs, ys, zs, idx):
    f32 = jnp.float32
    out = jax.ShapeDtypeStruct((NW, CHUNK), f32)
    fn = functools.partial(
        pl.kernel,
        mesh=plsc.VectorSubcoreMesh(core_axis_name="c", subcore_axis_name="s"),
        out_type=[out] * 3,
        scratch_types=[
            pltpu.VMEM((NCH, GW), jnp.int32),
            pltpu.VMEM((CHUNK,), f32),
            pltpu.VMEM((CHUNK,), f32),
            pltpu.VMEM((CHUNK,), f32),
            pltpu.SemaphoreType.DMA,
        ],
    )(_sc_gather_body)
    return fn(xs, ys, zs, idx)


def _mlp_body(gx_ref, gy_ref, gz_ref, cx_ref, cy_ref, cz_ref,
              w1g_ref, w1c_ref, b1_ref, w2_ref, b2_ref, out_ref):
    G = jnp.concatenate([gx_ref[0], gy_ref[0], gz_ref[0]], axis=0)
    Cm = jnp.concatenate([cx_ref[0], cy_ref[0], cz_ref[0]], axis=0)
    A = lax.dot_general(w1g_ref[...].astype(jnp.bfloat16),
                        G.astype(jnp.bfloat16),
                        (((1,), (0,)), ((), ())),
                        preferred_element_type=jnp.float32)
    T2 = lax.dot_general(w1c_ref[...].astype(jnp.bfloat16),
                         Cm.astype(jnp.bfloat16),
                         (((1,), (0,)), ((), ())),
                         preferred_element_type=jnp.float32)
    M = b1_ref[...] - T2
    w2 = w2_ref[...].astype(jnp.bfloat16)
    acc = None
    for k in range(K):
        h = jnp.maximum(A[:, k * S:(k + 1) * S] + M, 0.0)
        h2 = lax.dot_general(w2, h.astype(jnp.bfloat16),
                             (((1,), (0,)), ((), ())),
                             preferred_element_type=jnp.float32)
        acc = h2 if acc is None else jnp.maximum(acc, h2)
    out_ref[0] = acc + b2_ref[...]


def _run_mlp(gx3, gy3, gz3, cx3, cy3, cz3, W1g, W1c, b1r, W2, b2r):
    row = pl.BlockSpec((1, 1, S * K), lambda b: (b, 0, 0))
    crow = pl.BlockSpec((1, 1, S), lambda b: (b, 0, 0))
    return pl.pallas_call(
        _mlp_body,
        grid=(B,),
        in_specs=[row, row, row, crow, crow, crow,
                  pl.BlockSpec((64, 3), lambda b: (0, 0)),
                  pl.BlockSpec((64, 3), lambda b: (0, 0)),
                  pl.BlockSpec((64, 1), lambda b: (0, 0)),
                  pl.BlockSpec((128, 64), lambda b: (0, 0)),
                  pl.BlockSpec((128, 1), lambda b: (0, 0))],
        out_specs=pl.BlockSpec((1, 128, S), lambda b: (b, 0, 0)),
        out_shape=jax.ShapeDtypeStruct((B, 128, S), jnp.float32),
    )(gx3, gy3, gz3, cx3, cy3, cz3, W1g, W1c, b1r, W2, b2r)


def kernel(partial_cloud, kp, W1, b1, W2, b2):
    xyz = jnp.concatenate([partial_cloud, kp], axis=2)
    x = xyz[:, 0, :]
    y = xyz[:, 1, :]
    z = xyz[:, 2, :]

    cx, cy, cz, bn = _run_fps(x, y, z)

    idx = _run_knn(
        x.reshape(B, 1, N), y.reshape(B, 1, N), z.reshape(B, 1, N),
        bn.reshape(B, 1, N),
        cx.reshape(B * NQC, 1, QCHUNK), cy.reshape(B * NQC, 1, QCHUNK),
        cz.reshape(B * NQC, 1, QCHUNK))

    idx_sc = (idx.reshape(B, NQC, QCHUNK, K)
              .transpose(0, 3, 1, 2)
              .reshape(NW, NCH, GW))

    gx, gy, gz = _run_sc_gather(
        x.reshape(B * N), y.reshape(B * N), z.reshape(B * N), idx_sc)

    W1a = W1[:, :3]
    out = _run_mlp(
        gx.reshape(B, 1, S * K), gy.reshape(B, 1, S * K),
        gz.reshape(B, 1, S * K),
        cx.reshape(B, 1, S), cy.reshape(B, 1, S), cz.reshape(B, 1, S),
        W1a + W1[:, 3:], W1a, b1.reshape(64, 1), W2, b2.reshape(128, 1))
    return out


# speedup vs baseline: 22.8413x; 1.0533x over previous
import functools

import jax
import jax.numpy as jnp
from jax import lax
from jax.experimental import pallas as pl
from jax.experimental.pallas import tpu as pltpu
from jax.experimental.pallas import tpu_sc as plsc

B = 8
N = 8320
S = 512
K = 16
QCHUNK = 512
NQC = S // QCHUNK
NW = 32
CHUNK = (B * S * K) // NW
GW = 128
NCH = CHUNK // GW


def _fps_body(x_ref, y_ref, z_ref, cx_ref, cy_ref, cz_ref, bn_ref,
              xb_ref, yb_ref, zb_ref, dist_ref):
    x = x_ref[...]
    y = y_ref[...]
    z = z_ref[...]
    bn_ref[...] = (x * x + y * y) + z * z
    xb_ref[...] = x.astype(jnp.bfloat16)
    yb_ref[...] = y.astype(jnp.bfloat16)
    zb_ref[...] = z.astype(jnp.bfloat16)

    lane_iota = lax.broadcasted_iota(jnp.int32, (B, N), 1)
    sel_iota = lax.broadcasted_iota(jnp.int32, (B, S), 1)
    cx_ref[...] = jnp.zeros((B, S), jnp.float32)
    cy_ref[...] = jnp.zeros((B, S), jnp.float32)
    cz_ref[...] = jnp.zeros((B, S), jnp.float32)
    dist_ref[...] = jnp.full((B, N), 1e10, jnp.float32)

    def body(i, far):
        m2 = lane_iota == far
        cx = jnp.sum(jnp.where(m2, x, 0.0), axis=1, keepdims=True)
        cy = jnp.sum(jnp.where(m2, y, 0.0), axis=1, keepdims=True)
        cz = jnp.sum(jnp.where(m2, z, 0.0), axis=1, keepdims=True)
        onehot = sel_iota == i
        cx_ref[...] += jnp.where(onehot, cx, 0.0)
        cy_ref[...] += jnp.where(onehot, cy, 0.0)
        cz_ref[...] += jnp.where(onehot, cz, 0.0)
        dx = x - cx
        dy = y - cy
        dz = z - cz
        dd = (dx * dx + dy * dy) + dz * dz
        dist = jnp.minimum(dist_ref[...], dd)
        dist_ref[...] = dist
        far_new = jnp.argmax(dist, axis=1).astype(jnp.int32)[:, None]
        return far_new

    lax.fori_loop(0, S, body, jnp.zeros((B, 1), jnp.int32))


def _run_fps(x, y, z):
    return pl.pallas_call(
        _fps_body,
        out_shape=[
            jax.ShapeDtypeStruct((B, S), jnp.float32),
            jax.ShapeDtypeStruct((B, S), jnp.float32),
            jax.ShapeDtypeStruct((B, S), jnp.float32),
            jax.ShapeDtypeStruct((B, N), jnp.float32),
            jax.ShapeDtypeStruct((B, N), jnp.bfloat16),
            jax.ShapeDtypeStruct((B, N), jnp.bfloat16),
            jax.ShapeDtypeStruct((B, N), jnp.bfloat16),
        ],
        scratch_shapes=[pltpu.VMEM((B, N), jnp.float32)],
    )(x, y, z)


def _knn_body(x_ref, y_ref, z_ref, bn_ref, cx_ref, cy_ref, cz_ref,
              idx_ref, dist_ref):
    Xr = jnp.concatenate([x_ref[0], y_ref[0], z_ref[0]], axis=0)
    QT = jnp.concatenate([cx_ref[0], cy_ref[0], cz_ref[0]], axis=0)
    Q = jnp.transpose(QT, (1, 0))
    C = lax.dot_general(Q.astype(jnp.bfloat16), Xr,
                        (((1,), (0,)), ((), ())),
                        preferred_element_type=jnp.float32)
    qx = Q[:, 0:1]
    qy = Q[:, 1:2]
    qz = Q[:, 2:3]
    A = (qx * qx + qy * qy) + qz * qz
    dist_ref[...] = 2.0 * C - (A + bn_ref[0])

    iota = lax.broadcasted_iota(jnp.int32, (QCHUNK, N), 1)

    def pass_body(p, carry):
        prev, acc = carry
        s = jnp.where(iota == prev, -jnp.inf, dist_ref[...])
        dist_ref[...] = s
        idxc = jnp.argmax(s, axis=1).astype(jnp.int32)[:, None]
        kio = lax.broadcasted_iota(jnp.int32, (QCHUNK, K), 1)
        acc = jnp.where(kio == p, idxc, acc)
        return idxc, acc

    _, acc = lax.fori_loop(
        0, K, pass_body,
        (jnp.full((QCHUNK, 1), -1, jnp.int32),
         jnp.zeros((QCHUNK, K), jnp.int32)))
    idx_ref[0] = acc + pl.program_id(0) * N


def _run_knn(x3, y3, z3, bn3, cxr, cyr, czr):
    plane = pl.BlockSpec((1, 1, N), lambda b, q: (b, 0, 0))
    cspec = pl.BlockSpec((1, 1, QCHUNK), lambda b, q: (b * NQC + q, 0, 0))
    return pl.pallas_call(
        _knn_body,
        grid=(B, NQC),
        in_specs=[plane, plane, plane, plane, cspec, cspec, cspec],
        out_specs=pl.BlockSpec((1, QCHUNK, K), lambda b, q: (b * NQC + q, 0, 0)),
        out_shape=jax.ShapeDtypeStruct((B * NQC, QCHUNK, K), jnp.int32),
        scratch_shapes=[pltpu.VMEM((QCHUNK, N), jnp.float32)],
    )(x3, y3, z3, bn3, cxr, cyr, czr)


def _sc_gather_body(xs_hbm, ys_hbm, zs_hbm, idx_hbm,
                    ogx_hbm, ogy_hbm, ogz_hbm,
                    idxv, gxv, gyv, gzv, sem):
    wid = lax.axis_index("s") * 2 + lax.axis_index("c")
    pltpu.sync_copy(idx_hbm.at[wid], idxv)
    copies = []
    for j in range(NCH):
        iv = idxv.at[j]
        sl = pl.ds(j * GW, GW)
        copies.append(pltpu.async_copy(xs_hbm.at[iv], gxv.at[sl], sem))
        copies.append(pltpu.async_copy(ys_hbm.at[iv], gyv.at[sl], sem))
        copies.append(pltpu.async_copy(zs_hbm.at[iv], gzv.at[sl], sem))
    for c in copies:
        c.wait()
    pltpu.sync_copy(gxv, ogx_hbm.at[wid])
    pltpu.sync_copy(gyv, ogy_hbm.at[wid])
    pltpu.sync_copy(gzv, ogz_hbm.at[wid])


def _run_sc_gather(xs, ys, zs, idx):
    f32 = jnp.float32
    out = jax.ShapeDtypeStruct((NW, CHUNK), f32)
    fn = functools.partial(
        pl.kernel,
        mesh=plsc.VectorSubcoreMesh(core_axis_name="c", subcore_axis_name="s"),
        out_type=[out] * 3,
        scratch_types=[
            pltpu.VMEM((NCH, GW), jnp.int32),
            pltpu.VMEM((CHUNK,), f32),
            pltpu.VMEM((CHUNK,), f32),
            pltpu.VMEM((CHUNK,), f32),
            pltpu.SemaphoreType.DMA,
        ],
    )(_sc_gather_body)
    return fn(xs, ys, zs, idx)


def _mlp_body(gx_ref, gy_ref, gz_ref, cx_ref, cy_ref, cz_ref,
              w1g_ref, w1c_ref, b1_ref, w2_ref, b2_ref, out_ref):
    G = jnp.concatenate([gx_ref[0], gy_ref[0], gz_ref[0]], axis=0)
    Cm = jnp.concatenate([cx_ref[0], cy_ref[0], cz_ref[0]], axis=0)
    A = lax.dot_general(w1g_ref[...].astype(jnp.bfloat16),
                        G.astype(jnp.bfloat16),
                        (((1,), (0,)), ((), ())),
                        preferred_element_type=jnp.float32)
    T2 = lax.dot_general(w1c_ref[...].astype(jnp.bfloat16),
                         Cm.astype(jnp.bfloat16),
                         (((1,), (0,)), ((), ())),
                         preferred_element_type=jnp.float32)
    M = b1_ref[...] - T2
    w2 = w2_ref[...].astype(jnp.bfloat16)
    acc = None
    for k in range(K):
        h = jnp.maximum(A[:, k * S:(k + 1) * S] + M, 0.0)
        h2 = lax.dot_general(w2, h.astype(jnp.bfloat16),
                             (((1,), (0,)), ((), ())),
                             preferred_element_type=jnp.float32)
        acc = h2 if acc is None else jnp.maximum(acc, h2)
    out_ref[0] = acc + b2_ref[...]


def _run_mlp(gx3, gy3, gz3, cx3, cy3, cz3, W1g, W1c, b1r, W2, b2r):
    row = pl.BlockSpec((1, 1, S * K), lambda b: (b, 0, 0))
    crow = pl.BlockSpec((1, 1, S), lambda b: (b, 0, 0))
    return pl.pallas_call(
        _mlp_body,
        grid=(B,),
        in_specs=[row, row, row, crow, crow, crow,
                  pl.BlockSpec((64, 3), lambda b: (0, 0)),
                  pl.BlockSpec((64, 3), lambda b: (0, 0)),
                  pl.BlockSpec((64, 1), lambda b: (0, 0)),
                  pl.BlockSpec((128, 64), lambda b: (0, 0)),
                  pl.BlockSpec((128, 1), lambda b: (0, 0))],
        out_specs=pl.BlockSpec((1, 128, S), lambda b: (b, 0, 0)),
        out_shape=jax.ShapeDtypeStruct((B, 128, S), jnp.float32),
    )(gx3, gy3, gz3, cx3, cy3, cz3, W1g, W1c, b1r, W2, b2r)


def kernel(partial_cloud, kp, W1, b1, W2, b2):
    xyz = jnp.concatenate([partial_cloud, kp], axis=2)
    x = xyz[:, 0, :]
    y = xyz[:, 1, :]
    z = xyz[:, 2, :]

    cx, cy, cz, bn, xb, yb, zb = _run_fps(x, y, z)

    idx = _run_knn(
        xb.reshape(B, 1, N), yb.reshape(B, 1, N), zb.reshape(B, 1, N),
        bn.reshape(B, 1, N),
        cx.reshape(B * NQC, 1, QCHUNK), cy.reshape(B * NQC, 1, QCHUNK),
        cz.reshape(B * NQC, 1, QCHUNK))

    idx_sc = (idx.reshape(B, NQC, QCHUNK, K)
              .transpose(0, 3, 1, 2)
              .reshape(NW, NCH, GW))

    gx, gy, gz = _run_sc_gather(
        x.reshape(B * N), y.reshape(B * N), z.reshape(B * N), idx_sc)

    W1a = W1[:, :3]
    out = _run_mlp(
        gx.reshape(B, 1, S * K), gy.reshape(B, 1, S * K),
        gz.reshape(B, 1, S * K),
        cx.reshape(B, 1, S), cy.reshape(B, 1, S), cz.reshape(B, 1, S),
        W1a + W1[:, 3:], W1a, b1.reshape(64, 1), W2, b2.reshape(128, 1))
    return out
